# baseline (speedup 1.0000x reference)
import functools

import jax
import jax.numpy as jnp
from jax import lax
from jax.experimental import pallas as pl
from jax.experimental.pallas import tpu as pltpu
from jax.experimental.pallas import tpu_sc as plsc

E = 8
TOPK = 2
H = 1024
I = 3584
T = 2048
TK = T * TOPK
BM = 128
NB = TK // BM
NITEMS = NB + E - 1
BI = 896
NI = I // BI


def _router_body(x_ref, wg_ref, idx_ref, w_ref):
    x = x_ref[...]
    wg = wg_ref[...]
    logits = lax.dot_general(x, wg, (((1,), (1,)), ((), ())),
                             preferred_element_type=jnp.float32)
    eio = lax.broadcasted_iota(jnp.int32, (T, E), 1)
    m1 = jnp.max(logits, axis=1, keepdims=True)
    i1 = jnp.min(jnp.where(logits == m1, eio, E), axis=1, keepdims=True)
    l2 = jnp.where(eio == i1, -jnp.inf, logits)
    m2 = jnp.max(l2, axis=1, keepdims=True)
    i2 = jnp.min(jnp.where(l2 == m2, eio, E), axis=1, keepdims=True)
    w1 = 1.0 / (1.0 + jnp.exp(m2 - m1))
    w2 = 1.0 - w1
    zi = jnp.zeros((T, 126), jnp.int32)
    zf = jnp.zeros((T, 126), jnp.float32)
    idx_ref[...] = jnp.concatenate([i1, i2, zi], axis=1)
    w_ref[...] = jnp.concatenate([w1, w2, zf], axis=1)


def _router(x, wg):
    return pl.pallas_call(
        _router_body,
        out_shape=(jax.ShapeDtypeStruct((T, 128), jnp.int32),
                   jax.ShapeDtypeStruct((T, 128), jnp.float32)),
    )(x, wg)


def _plan_dispatch(i1, i2):
    e_flat = jnp.concatenate([i1, i2]).astype(jnp.int32)
    oh = (e_flat[:, None] ==
          jnp.arange(E, dtype=jnp.int32)[None, :]).astype(jnp.int32)
    csum = jnp.cumsum(oh, axis=0)
    counts = csum[-1]
    gs = jnp.concatenate([jnp.zeros((1,), jnp.int32),
                          jnp.cumsum(counts)]).astype(jnp.int32)
    rank = jnp.sum(oh * csum, axis=1) - 1
    base_e = jnp.sum(oh * gs[None, :E], axis=1)
    inv = base_e + rank
    pos0, pos1 = inv[:T], inv[T:]
    tok = jnp.concatenate([jnp.arange(T, dtype=jnp.int32)] * 2)
    sorted_t = jnp.zeros((TK,), jnp.int32).at[inv].set(tok)
    b = jnp.arange(NB, dtype=jnp.int32)
    blo = (b * BM)[:, None]
    plo = jnp.maximum(blo, gs[None, :E])
    phi = jnp.minimum(blo + BM, gs[None, 1:])
    valid = phi > plo
    slot = jnp.where(valid,
                     jnp.cumsum(valid.ravel()).reshape(NB, E).astype(jnp.int32) - 1,
                     NITEMS)
    pad_e = jnp.max(e_flat)
    sl = slot.ravel()
    bb = jnp.broadcast_to(b[:, None], (NB, E)).ravel()
    ee = jnp.broadcast_to(jnp.arange(E, dtype=jnp.int32)[None, :], (NB, E)).ravel()
    item_rb = jnp.full((NITEMS,), NB - 1, jnp.int32).at[sl].set(bb, mode='drop')
    item_e = jnp.broadcast_to(pad_e, (NITEMS,)).astype(jnp.int32).at[sl].set(ee, mode='drop')
    item_lo = jnp.zeros((NITEMS,), jnp.int32).at[sl].set(plo.ravel().astype(jnp.int32), mode='drop')
    item_hi = jnp.zeros((NITEMS,), jnp.int32).at[sl].set(phi.ravel().astype(jnp.int32), mode='drop')
    return sorted_t, pos0, pos1, item_rb, item_e, item_lo, item_hi


def _sc_gather_rows(table, idx):
    v, d = table.shape
    bsz = idx.shape[0]
    info = plsc.get_sparse_core_info()
    nw = info.num_cores * info.num_subcores
    rows_per_w = bsz // nw
    ch = min(64, rows_per_w)
    nch = rows_per_w // ch
    mesh = plsc.VectorSubcoreMesh(core_axis_name="c", subcore_axis_name="s")

    @functools.partial(
        pl.kernel, mesh=mesh,
        out_type=jax.ShapeDtypeStruct((bsz, d), jnp.float32),
        scratch_types=[pltpu.VMEM((ch,), jnp.int32),
                       pltpu.VMEM((ch, d), jnp.float32),
                       pltpu.SemaphoreType.DMA],
    )
    def k(table_hbm, idx_hbm, out_hbm, idx_v, rows_v, sem):
        wid = lax.axis_index("s") * info.num_cores + lax.axis_index("c")
        for c in range(nch):
            base = wid * rows_per_w + c * ch
            pltpu.sync_copy(idx_hbm.at[pl.ds(base, ch)], idx_v)
            pltpu.async_copy(table_hbm.at[idx_v], rows_v, sem).wait()
            pltpu.sync_copy(rows_v, out_hbm.at[pl.ds(base, ch)])

    return k(table, idx)


def _ffn_body(rb_ref, e_ref, lo_ref, hi_ref,
              xs_ref, w1_ref, w3_ref, w2_ref, out_ref,
              w1b_ref, w3b_ref, w2b_ref):
    i = pl.program_id(0)
    j = pl.program_id(1)
    base = rb_ref[j] * BM

    jm = jnp.maximum(j - 1, 0)
    changed = (j == 0) | (e_ref[j] != e_ref[jm])

    @pl.when(changed)
    def _():
        w1b_ref[...] = w1_ref[0].astype(jnp.bfloat16)
        w3b_ref[...] = w3_ref[0].astype(jnp.bfloat16)
        w2b_ref[...] = w2_ref[0].astype(jnp.bfloat16)

    x = xs_ref[pl.ds(base, BM), :].astype(jnp.bfloat16)
    g = lax.dot_general(x, w1b_ref[...], (((1,), (1,)), ((), ())),
                        preferred_element_type=jnp.float32)
    u = lax.dot_general(x, w3b_ref[...], (((1,), (1,)), ((), ())),
                        preferred_element_type=jnp.float32)
    act = g * (1.0 / (1.0 + jnp.exp(-g))) * u
    part = lax.dot_general(act.astype(jnp.bfloat16), w2b_ref[...],
                           (((1,), (1,)), ((), ())),
                           preferred_element_type=jnp.float32)

    rows = base + lax.broadcasted_iota(jnp.int32, (BM, 1), 0)
    mask = (rows >= lo_ref[j]) & (rows < hi_ref[j])

    @pl.when(i == 0)
    def _():
        cur = out_ref[pl.ds(base, BM), :]
        out_ref[pl.ds(base, BM), :] = jnp.where(mask, part, cur)

    @pl.when(i > 0)
    def _():
        cur = out_ref[pl.ds(base, BM), :]
        out_ref[pl.ds(base, BM), :] = jnp.where(mask, cur + part, cur)


def _ffn(item_rb, item_e, item_lo, item_hi, xs, w1, w3, w2):
    grid_spec = pltpu.PrefetchScalarGridSpec(
        num_scalar_prefetch=4,
        grid=(NI, NITEMS),
        in_specs=[
            pl.BlockSpec((TK, H), lambda i, j, rb, e, lo, hi: (0, 0)),
            pl.BlockSpec((1, BI, H), lambda i, j, rb, e, lo, hi: (e[j], i, 0)),
            pl.BlockSpec((1, BI, H), lambda i, j, rb, e, lo, hi: (e[j], i, 0)),
            pl.BlockSpec((1, H, BI), lambda i, j, rb, e, lo, hi: (e[j], 0, i)),
        ],
        out_specs=pl.BlockSpec((TK, H), lambda i, j, rb, e, lo, hi: (0, 0)),
        scratch_shapes=[pltpu.VMEM((BI, H), jnp.bfloat16),
                        pltpu.VMEM((BI, H), jnp.bfloat16),
                        pltpu.VMEM((H, BI), jnp.bfloat16)],
    )
    return pl.pallas_call(
        _ffn_body,
        grid_spec=grid_spec,
        out_shape=jax.ShapeDtypeStruct((TK, H), jnp.float32),
        compiler_params=pltpu.CompilerParams(
            vmem_limit_bytes=60 * 1024 * 1024),
    )(item_rb, item_e, item_lo, item_hi, xs, w1, w3, w2)


def _combine_body(z0_ref, z1_ref, w_ref, out_ref):
    w0 = w_ref[:, 0:1]
    w1 = w_ref[:, 1:2]
    out_ref[...] = w0 * z0_ref[...] + w1 * z1_ref[...]


def _combine(z, w_pad):
    nblk = T // BM
    return pl.pallas_call(
        _combine_body,
        grid=(nblk,),
        in_specs=[
            pl.BlockSpec((BM, H), lambda i: (i, 0)),
            pl.BlockSpec((BM, H), lambda i, _n=nblk: (i + _n, 0)),
            pl.BlockSpec((BM, 128), lambda i: (i, 0)),
        ],
        out_specs=pl.BlockSpec((BM, H), lambda i: (i, 0)),
        out_shape=jax.ShapeDtypeStruct((T, H), jnp.float32),
    )(z, z, w_pad)


def kernel(hidden_states, Wg, W1, W3, W2):
    x = hidden_states.reshape(T, H)
    idx_pad, w_pad = _router(x, Wg)
    i1, i2 = idx_pad[:, 0], idx_pad[:, 1]
    sorted_t, pos0, pos1, item_rb, item_e, item_lo, item_hi = \
        _plan_dispatch(i1, i2)
    xs = _sc_gather_rows(x, sorted_t)
    y_sorted = _ffn(item_rb, item_e, item_lo, item_hi, xs, W1, W3, W2)
    z = _sc_gather_rows(y_sorted, jnp.concatenate([pos0, pos1]))
    out = _combine(z, w_pad)
    return out.reshape(hidden_states.shape)


# speedup vs baseline: 1.4326x; 1.4326x over previous
import functools

import jax
import jax.numpy as jnp
from jax import lax
from jax.experimental import pallas as pl
from jax.experimental.pallas import tpu as pltpu
from jax.experimental.pallas import tpu_sc as plsc

E = 8
TOPK = 2
H = 1024
I = 3584
T = 2048
TK = T * TOPK
BM = 256
NB = TK // BM
NITEMS = NB + E - 1
BI = 896
NI = I // BI


def _router_body(x_ref, wg_ref, idx_ref, w_ref):
    x = x_ref[...]
    wg = wg_ref[...]
    logits = lax.dot_general(x, wg, (((1,), (1,)), ((), ())),
                             preferred_element_type=jnp.float32)
    eio = lax.broadcasted_iota(jnp.int32, (T, E), 1)
    m1 = jnp.max(logits, axis=1, keepdims=True)
    i1 = jnp.min(jnp.where(logits == m1, eio, E), axis=1, keepdims=True)
    l2 = jnp.where(eio == i1, -jnp.inf, logits)
    m2 = jnp.max(l2, axis=1, keepdims=True)
    i2 = jnp.min(jnp.where(l2 == m2, eio, E), axis=1, keepdims=True)
    w1 = 1.0 / (1.0 + jnp.exp(m2 - m1))
    w2 = 1.0 - w1
    zi = jnp.zeros((T, 126), jnp.int32)
    zf = jnp.zeros((T, 126), jnp.float32)
    idx_ref[...] = jnp.concatenate([i1, i2, zi], axis=1)
    w_ref[...] = jnp.concatenate([w1, w2, zf], axis=1)


def _router(x, wg):
    return pl.pallas_call(
        _router_body,
        out_shape=(jax.ShapeDtypeStruct((T, 128), jnp.int32),
                   jax.ShapeDtypeStruct((T, 128), jnp.float32)),
    )(x, wg)


def _plan_dispatch(i1, i2):
    e_flat = jnp.concatenate([i1, i2]).astype(jnp.int32)
    oh = (e_flat[:, None] ==
          jnp.arange(E, dtype=jnp.int32)[None, :]).astype(jnp.int32)
    csum = jnp.cumsum(oh, axis=0)
    counts = csum[-1]
    gs = jnp.concatenate([jnp.zeros((1,), jnp.int32),
                          jnp.cumsum(counts)]).astype(jnp.int32)
    rank = jnp.sum(oh * csum, axis=1) - 1
    base_e = jnp.sum(oh * gs[None, :E], axis=1)
    inv = base_e + rank
    pos0, pos1 = inv[:T], inv[T:]
    tok = jnp.concatenate([jnp.arange(T, dtype=jnp.int32)] * 2)
    sorted_t = jnp.zeros((TK,), jnp.int32).at[inv].set(tok)
    b = jnp.arange(NB, dtype=jnp.int32)
    blo = (b * BM)[:, None]
    plo = jnp.maximum(blo, gs[None, :E])
    phi = jnp.minimum(blo + BM, gs[None, 1:])
    valid = phi > plo
    slot = jnp.where(valid,
                     jnp.cumsum(valid.ravel()).reshape(NB, E).astype(jnp.int32) - 1,
                     NITEMS)
    pad_e = jnp.max(e_flat)
    sl = slot.ravel()
    bb = jnp.broadcast_to(b[:, None], (NB, E)).ravel()
    ee = jnp.broadcast_to(jnp.arange(E, dtype=jnp.int32)[None, :], (NB, E)).ravel()
    item_rb = jnp.full((NITEMS,), NB - 1, jnp.int32).at[sl].set(bb, mode='drop')
    item_e = jnp.broadcast_to(pad_e, (NITEMS,)).astype(jnp.int32).at[sl].set(ee, mode='drop')
    item_lo = jnp.zeros((NITEMS,), jnp.int32).at[sl].set(plo.ravel().astype(jnp.int32), mode='drop')
    item_hi = jnp.zeros((NITEMS,), jnp.int32).at[sl].set(phi.ravel().astype(jnp.int32), mode='drop')
    return sorted_t, pos0, pos1, item_rb, item_e, item_lo, item_hi


def _sc_gather_rows(table, idx):
    v, d = table.shape
    bsz = idx.shape[0]
    info = plsc.get_sparse_core_info()
    nw = info.num_cores * info.num_subcores
    rows_per_w = bsz // nw
    ch = min(64, rows_per_w)
    nch = rows_per_w // ch
    mesh = plsc.VectorSubcoreMesh(core_axis_name="c", subcore_axis_name="s")

    @functools.partial(
        pl.kernel, mesh=mesh,
        out_type=jax.ShapeDtypeStruct((bsz, d), jnp.float32),
        scratch_types=[pltpu.VMEM((ch,), jnp.int32),
                       pltpu.VMEM((ch, d), jnp.float32),
                       pltpu.SemaphoreType.DMA],
    )
    def k(table_hbm, idx_hbm, out_hbm, idx_v, rows_v, sem):
        wid = lax.axis_index("s") * info.num_cores + lax.axis_index("c")
        for c in range(nch):
            base = wid * rows_per_w + c * ch
            pltpu.sync_copy(idx_hbm.at[pl.ds(base, ch)], idx_v)
            pltpu.async_copy(table_hbm.at[idx_v], rows_v, sem).wait()
            pltpu.sync_copy(rows_v, out_hbm.at[pl.ds(base, ch)])

    return k(table, idx)


def _ffn_body(rb_ref, e_ref, lo_ref, hi_ref,
              xs_ref, w1_ref, w3_ref, w2_ref, out_ref):
    i = pl.program_id(0)
    j = pl.program_id(1)
    base = rb_ref[j] * BM
    x = xs_ref[pl.ds(base, BM), :]
    w1 = w1_ref[0]
    w3 = w3_ref[0]
    g = lax.dot_general(x, w1, (((1,), (1,)), ((), ())),
                        preferred_element_type=jnp.float32)
    u = lax.dot_general(x, w3, (((1,), (1,)), ((), ())),
                        preferred_element_type=jnp.float32)
    act = g * (1.0 / (1.0 + jnp.exp(-g))) * u
    w2 = w2_ref[0]
    part = lax.dot_general(act, w2, (((1,), (1,)), ((), ())),
                           preferred_element_type=jnp.float32)

    rows = base + lax.broadcasted_iota(jnp.int32, (BM, 1), 0)
    mask = (rows >= lo_ref[j]) & (rows < hi_ref[j])

    @pl.when(i == 0)
    def _():
        cur = out_ref[pl.ds(base, BM), :]
        out_ref[pl.ds(base, BM), :] = jnp.where(mask, part, cur)

    @pl.when(i > 0)
    def _():
        cur = out_ref[pl.ds(base, BM), :]
        out_ref[pl.ds(base, BM), :] = jnp.where(mask, cur + part, cur)


def _ffn(item_rb, item_e, item_lo, item_hi, xs, w1, w3, w2):
    grid_spec = pltpu.PrefetchScalarGridSpec(
        num_scalar_prefetch=4,
        grid=(NI, NITEMS),
        in_specs=[
            pl.BlockSpec((TK, H), lambda i, j, rb, e, lo, hi: (0, 0)),
            pl.BlockSpec((1, BI, H), lambda i, j, rb, e, lo, hi: (e[j], i, 0)),
            pl.BlockSpec((1, BI, H), lambda i, j, rb, e, lo, hi: (e[j], i, 0)),
            pl.BlockSpec((1, H, BI), lambda i, j, rb, e, lo, hi: (e[j], 0, i)),
        ],
        out_specs=pl.BlockSpec((TK, H), lambda i, j, rb, e, lo, hi: (0, 0)),
    )
    return pl.pallas_call(
        _ffn_body,
        grid_spec=grid_spec,
        out_shape=jax.ShapeDtypeStruct((TK, H), jnp.float32),
        compiler_params=pltpu.CompilerParams(
            vmem_limit_bytes=60 * 1024 * 1024),
    )(item_rb, item_e, item_lo, item_hi, xs, w1, w3, w2)


def _combine_body(z0_ref, z1_ref, w_ref, out_ref):
    w0 = w_ref[:, 0:1]
    w1 = w_ref[:, 1:2]
    out_ref[...] = w0 * z0_ref[...] + w1 * z1_ref[...]


def _combine(z, w_pad):
    nblk = T // BM
    return pl.pallas_call(
        _combine_body,
        grid=(nblk,),
        in_specs=[
            pl.BlockSpec((BM, H), lambda i: (i, 0)),
            pl.BlockSpec((BM, H), lambda i, _n=nblk: (i + _n, 0)),
            pl.BlockSpec((BM, 128), lambda i: (i, 0)),
        ],
        out_specs=pl.BlockSpec((BM, H), lambda i: (i, 0)),
        out_shape=jax.ShapeDtypeStruct((T, H), jnp.float32),
    )(z, z, w_pad)


def kernel(hidden_states, Wg, W1, W3, W2):
    x = hidden_states.reshape(T, H)
    idx_pad, w_pad = _router(x, Wg)
    i1, i2 = idx_pad[:, 0], idx_pad[:, 1]
    sorted_t, pos0, pos1, item_rb, item_e, item_lo, item_hi = \
        _plan_dispatch(i1, i2)
    xs = _sc_gather_rows(x, sorted_t)
    y_sorted = _ffn(item_rb, item_e, item_lo, item_hi, xs, W1, W3, W2)
    z = _sc_gather_rows(y_sorted, jnp.concatenate([pos0, pos1]))
    out = _combine(z, w_pad)
    return out.reshape(hidden_states.shape)


# speedup vs baseline: 1.5119x; 1.0553x over previous
import functools

import jax
import jax.numpy as jnp
from jax import lax
from jax.experimental import pallas as pl
from jax.experimental.pallas import tpu as pltpu
from jax.experimental.pallas import tpu_sc as plsc

E = 8
TOPK = 2
H = 1024
I = 3584
T = 2048
TK = T * TOPK
BM = 512
NB = TK // BM
NITEMS = NB + E - 1
BI = 896
NI = I // BI


def _router_body(x_ref, wg_ref, idx_ref, w_ref):
    x = x_ref[...]
    wg = wg_ref[...]
    logits = lax.dot_general(x, wg, (((1,), (1,)), ((), ())),
                             preferred_element_type=jnp.float32)
    eio = lax.broadcasted_iota(jnp.int32, (T, E), 1)
    m1 = jnp.max(logits, axis=1, keepdims=True)
    i1 = jnp.min(jnp.where(logits == m1, eio, E), axis=1, keepdims=True)
    l2 = jnp.where(eio == i1, -jnp.inf, logits)
    m2 = jnp.max(l2, axis=1, keepdims=True)
    i2 = jnp.min(jnp.where(l2 == m2, eio, E), axis=1, keepdims=True)
    w1 = 1.0 / (1.0 + jnp.exp(m2 - m1))
    w2 = 1.0 - w1
    zi = jnp.zeros((T, 126), jnp.int32)
    zf = jnp.zeros((T, 126), jnp.float32)
    idx_ref[...] = jnp.concatenate([i1, i2, zi], axis=1)
    w_ref[...] = jnp.concatenate([w1, w2, zf], axis=1)


def _router(x, wg):
    return pl.pallas_call(
        _router_body,
        out_shape=(jax.ShapeDtypeStruct((T, 128), jnp.int32),
                   jax.ShapeDtypeStruct((T, 128), jnp.float32)),
    )(x, wg)


def _plan_dispatch(i1, i2):
    e_flat = jnp.concatenate([i1, i2]).astype(jnp.int32)
    oh = (e_flat[:, None] ==
          jnp.arange(E, dtype=jnp.int32)[None, :]).astype(jnp.int32)
    csum = jnp.cumsum(oh, axis=0)
    counts = csum[-1]
    gs = jnp.concatenate([jnp.zeros((1,), jnp.int32),
                          jnp.cumsum(counts)]).astype(jnp.int32)
    rank = jnp.sum(oh * csum, axis=1) - 1
    base_e = jnp.sum(oh * gs[None, :E], axis=1)
    inv = base_e + rank
    pos0, pos1 = inv[:T], inv[T:]
    tok = jnp.concatenate([jnp.arange(T, dtype=jnp.int32)] * 2)
    sorted_t = jnp.zeros((TK,), jnp.int32).at[inv].set(tok)
    b = jnp.arange(NB, dtype=jnp.int32)
    blo = (b * BM)[:, None]
    plo = jnp.maximum(blo, gs[None, :E])
    phi = jnp.minimum(blo + BM, gs[None, 1:])
    valid = phi > plo
    slot = jnp.where(valid,
                     jnp.cumsum(valid.ravel()).reshape(NB, E).astype(jnp.int32) - 1,
                     NITEMS)
    pad_e = jnp.max(e_flat)
    sl = slot.ravel()
    bb = jnp.broadcast_to(b[:, None], (NB, E)).ravel()
    ee = jnp.broadcast_to(jnp.arange(E, dtype=jnp.int32)[None, :], (NB, E)).ravel()
    item_rb = jnp.full((NITEMS,), NB - 1, jnp.int32).at[sl].set(bb, mode='drop')
    item_e = jnp.broadcast_to(pad_e, (NITEMS,)).astype(jnp.int32).at[sl].set(ee, mode='drop')
    item_lo = jnp.zeros((NITEMS,), jnp.int32).at[sl].set(plo.ravel().astype(jnp.int32), mode='drop')
    item_hi = jnp.zeros((NITEMS,), jnp.int32).at[sl].set(phi.ravel().astype(jnp.int32), mode='drop')
    return sorted_t, pos0, pos1, item_rb, item_e, item_lo, item_hi


def _sc_gather_rows(table, idx):
    v, d = table.shape
    bsz = idx.shape[0]
    info = plsc.get_sparse_core_info()
    nw = info.num_cores * info.num_subcores
    rows_per_w = bsz // nw
    ch = min(64, rows_per_w)
    nch = rows_per_w // ch
    mesh = plsc.VectorSubcoreMesh(core_axis_name="c", subcore_axis_name="s")

    @functools.partial(
        pl.kernel, mesh=mesh,
        out_type=jax.ShapeDtypeStruct((bsz, d), jnp.float32),
        scratch_types=[pltpu.VMEM((ch,), jnp.int32),
                       pltpu.VMEM((ch, d), jnp.float32),
                       pltpu.SemaphoreType.DMA],
    )
    def k(table_hbm, idx_hbm, out_hbm, idx_v, rows_v, sem):
        wid = lax.axis_index("s") * info.num_cores + lax.axis_index("c")
        for c in range(nch):
            base = wid * rows_per_w + c * ch
            pltpu.sync_copy(idx_hbm.at[pl.ds(base, ch)], idx_v)
            pltpu.async_copy(table_hbm.at[idx_v], rows_v, sem).wait()
            pltpu.sync_copy(rows_v, out_hbm.at[pl.ds(base, ch)])

    return k(table, idx)


def _ffn_body(rb_ref, e_ref, lo_ref, hi_ref,
              xs_ref, w1_ref, w3_ref, w2_ref, out_ref):
    i = pl.program_id(0)
    j = pl.program_id(1)
    base = rb_ref[j] * BM
    x = xs_ref[pl.ds(base, BM), :]
    w1 = w1_ref[0]
    w3 = w3_ref[0]
    g = lax.dot_general(x, w1, (((1,), (1,)), ((), ())),
                        preferred_element_type=jnp.float32)
    u = lax.dot_general(x, w3, (((1,), (1,)), ((), ())),
                        preferred_element_type=jnp.float32)
    act = g * (1.0 / (1.0 + jnp.exp(-g))) * u
    w2 = w2_ref[0]
    part = lax.dot_general(act, w2, (((1,), (1,)), ((), ())),
                           preferred_element_type=jnp.float32)

    rows = base + lax.broadcasted_iota(jnp.int32, (BM, 1), 0)
    mask = (rows >= lo_ref[j]) & (rows < hi_ref[j])

    @pl.when(i == 0)
    def _():
        cur = out_ref[pl.ds(base, BM), :]
        out_ref[pl.ds(base, BM), :] = jnp.where(mask, part, cur)

    @pl.when(i > 0)
    def _():
        cur = out_ref[pl.ds(base, BM), :]
        out_ref[pl.ds(base, BM), :] = jnp.where(mask, cur + part, cur)


def _ffn(item_rb, item_e, item_lo, item_hi, xs, w1, w3, w2):
    grid_spec = pltpu.PrefetchScalarGridSpec(
        num_scalar_prefetch=4,
        grid=(NI, NITEMS),
        in_specs=[
            pl.BlockSpec((TK, H), lambda i, j, rb, e, lo, hi: (0, 0)),
            pl.BlockSpec((1, BI, H), lambda i, j, rb, e, lo, hi: (e[j], i, 0)),
            pl.BlockSpec((1, BI, H), lambda i, j, rb, e, lo, hi: (e[j], i, 0)),
            pl.BlockSpec((1, H, BI), lambda i, j, rb, e, lo, hi: (e[j], 0, i)),
        ],
        out_specs=pl.BlockSpec((TK, H), lambda i, j, rb, e, lo, hi: (0, 0)),
    )
    return pl.pallas_call(
        _ffn_body,
        grid_spec=grid_spec,
        out_shape=jax.ShapeDtypeStruct((TK, H), jnp.float32),
        compiler_params=pltpu.CompilerParams(
            vmem_limit_bytes=60 * 1024 * 1024),
    )(item_rb, item_e, item_lo, item_hi, xs, w1, w3, w2)


def _combine_body(z0_ref, z1_ref, w_ref, out_ref):
    w0 = w_ref[:, 0:1]
    w1 = w_ref[:, 1:2]
    out_ref[...] = w0 * z0_ref[...] + w1 * z1_ref[...]


def _combine(z, w_pad):
    nblk = T // BM
    return pl.pallas_call(
        _combine_body,
        grid=(nblk,),
        in_specs=[
            pl.BlockSpec((BM, H), lambda i: (i, 0)),
            pl.BlockSpec((BM, H), lambda i, _n=nblk: (i + _n, 0)),
            pl.BlockSpec((BM, 128), lambda i: (i, 0)),
        ],
        out_specs=pl.BlockSpec((BM, H), lambda i: (i, 0)),
        out_shape=jax.ShapeDtypeStruct((T, H), jnp.float32),
    )(z, z, w_pad)


def kernel(hidden_states, Wg, W1, W3, W2):
    x = hidden_states.reshape(T, H)
    idx_pad, w_pad = _router(x, Wg)
    i1, i2 = idx_pad[:, 0], idx_pad[:, 1]
    sorted_t, pos0, pos1, item_rb, item_e, item_lo, item_hi = \
        _plan_dispatch(i1, i2)
    xs = _sc_gather_rows(x, sorted_t)
    y_sorted = _ffn(item_rb, item_e, item_lo, item_hi, xs, W1, W3, W2)
    z = _sc_gather_rows(y_sorted, jnp.concatenate([pos0, pos1]))
    out = _combine(z, w_pad)
    return out.reshape(hidden_states.shape)


# speedup vs baseline: 1.6549x; 1.0946x over previous
import functools

import jax
import jax.numpy as jnp
from jax import lax
from jax.experimental import pallas as pl
from jax.experimental.pallas import tpu as pltpu
from jax.experimental.pallas import tpu_sc as plsc

E = 8
TOPK = 2
H = 1024
I = 3584
T = 2048
TK = T * TOPK
BM = 512
BMS = BM - 8
SMAX = -(-T // BMS)
NITEMS = 16
BI = 896
NI = I // BI


def _router_body(x_ref, wg_ref, idx_ref, w_ref):
    x = x_ref[...]
    wg = wg_ref[...]
    logits = lax.dot_general(x, wg, (((1,), (1,)), ((), ())),
                             preferred_element_type=jnp.float32)
    eio = lax.broadcasted_iota(jnp.int32, (T, E), 1)
    m1 = jnp.max(logits, axis=1, keepdims=True)
    i1 = jnp.min(jnp.where(logits == m1, eio, E), axis=1, keepdims=True)
    l2 = jnp.where(eio == i1, -jnp.inf, logits)
    m2 = jnp.max(l2, axis=1, keepdims=True)
    i2 = jnp.min(jnp.where(l2 == m2, eio, E), axis=1, keepdims=True)
    w1 = 1.0 / (1.0 + jnp.exp(m2 - m1))
    w2 = 1.0 - w1
    zi = jnp.zeros((T, 126), jnp.int32)
    zf = jnp.zeros((T, 126), jnp.float32)
    idx_ref[...] = jnp.concatenate([i1, i2, zi], axis=1)
    w_ref[...] = jnp.concatenate([w1, w2, zf], axis=1)


def _router(x, wg):
    return pl.pallas_call(
        _router_body,
        out_shape=(jax.ShapeDtypeStruct((T, 128), jnp.int32),
                   jax.ShapeDtypeStruct((T, 128), jnp.float32)),
    )(x, wg)


def _plan_dispatch(i1, i2):
    e_flat = jnp.concatenate([i1, i2]).astype(jnp.int32)
    oh = (e_flat[:, None] ==
          jnp.arange(E, dtype=jnp.int32)[None, :]).astype(jnp.int32)
    csum = jnp.cumsum(oh, axis=0)
    counts = csum[-1]
    gs = jnp.concatenate([jnp.zeros((1,), jnp.int32),
                          jnp.cumsum(counts)]).astype(jnp.int32)
    rank = jnp.sum(oh * csum, axis=1) - 1
    base_e = jnp.sum(oh * gs[None, :E], axis=1)
    inv = base_e + rank
    pos0, pos1 = inv[:T], inv[T:]
    tok = jnp.concatenate([jnp.arange(T, dtype=jnp.int32)] * 2)
    sorted_t = jnp.zeros((TK,), jnp.int32).at[inv].set(tok)
    ss = jnp.arange(SMAX, dtype=jnp.int32)[None, :]
    plo = gs[:E, None] + ss * BMS
    phi = jnp.minimum(gs[1:, None], plo + BMS)
    valid = plo < gs[1:, None]
    pbase = jnp.minimum((plo // 8) * 8, TK - BM)
    slot = jnp.where(valid,
                     jnp.cumsum(valid.ravel()).reshape(E, SMAX).astype(jnp.int32) - 1,
                     NITEMS)
    pad_e = jnp.max(e_flat)
    sl = slot.ravel()
    ee = jnp.broadcast_to(jnp.arange(E, dtype=jnp.int32)[:, None], (E, SMAX)).ravel()
    item_base = jnp.zeros((NITEMS,), jnp.int32).at[sl].set(pbase.ravel() // 8,
                                                           mode='drop')
    item_e = jnp.broadcast_to(pad_e, (NITEMS,)).astype(jnp.int32).at[sl].set(ee, mode='drop')
    item_lo = jnp.zeros((NITEMS,), jnp.int32).at[sl].set(plo.ravel(), mode='drop')
    item_hi = jnp.zeros((NITEMS,), jnp.int32).at[sl].set(phi.ravel(), mode='drop')
    return sorted_t, pos0, pos1, item_base, item_e, item_lo, item_hi


def _sc_gather_rows(table, idx):
    v, d = table.shape
    bsz = idx.shape[0]
    info = plsc.get_sparse_core_info()
    nw = info.num_cores * info.num_subcores
    rows_per_w = bsz // nw
    ch = min(64, rows_per_w)
    nch = rows_per_w // ch
    mesh = plsc.VectorSubcoreMesh(core_axis_name="c", subcore_axis_name="s")

    @functools.partial(
        pl.kernel, mesh=mesh,
        out_type=jax.ShapeDtypeStruct((bsz, d), jnp.float32),
        scratch_types=[pltpu.VMEM((ch,), jnp.int32),
                       pltpu.VMEM((ch, d), jnp.float32),
                       pltpu.SemaphoreType.DMA],
    )
    def k(table_hbm, idx_hbm, out_hbm, idx_v, rows_v, sem):
        wid = lax.axis_index("s") * info.num_cores + lax.axis_index("c")
        for c in range(nch):
            base = wid * rows_per_w + c * ch
            pltpu.sync_copy(idx_hbm.at[pl.ds(base, ch)], idx_v)
            pltpu.async_copy(table_hbm.at[idx_v], rows_v, sem).wait()
            pltpu.sync_copy(rows_v, out_hbm.at[pl.ds(base, ch)])

    return k(table, idx)


def _ffn_body(base_ref, e_ref, lo_ref, hi_ref,
              xs_ref, w1_ref, w3_ref, w2_ref, out_ref):
    i = pl.program_id(0)
    j = pl.program_id(1)

    @pl.when(hi_ref[j] > lo_ref[j])
    def _():
        base = base_ref[j] * 8
        x = xs_ref[pl.ds(base, BM), :]
        w1 = w1_ref[0]
        w3 = w3_ref[0]
        g = lax.dot_general(x, w1, (((1,), (1,)), ((), ())),
                            preferred_element_type=jnp.float32)
        u = lax.dot_general(x, w3, (((1,), (1,)), ((), ())),
                            preferred_element_type=jnp.float32)
        act = g * (1.0 / (1.0 + jnp.exp(-g))) * u
        w2 = w2_ref[0]
        part = lax.dot_general(act, w2, (((1,), (1,)), ((), ())),
                               preferred_element_type=jnp.float32)

        rows = base + lax.broadcasted_iota(jnp.int32, (BM, 1), 0)
        mask = (rows >= lo_ref[j]) & (rows < hi_ref[j])
        cur = out_ref[pl.ds(base, BM), :]

        @pl.when(i == 0)
        def _():
            out_ref[pl.ds(base, BM), :] = jnp.where(mask, part, cur)

        @pl.when(i > 0)
        def _():
            out_ref[pl.ds(base, BM), :] = jnp.where(mask, cur + part, cur)


def _ffn(item_rb, item_e, item_lo, item_hi, xs, w1, w3, w2):
    grid_spec = pltpu.PrefetchScalarGridSpec(
        num_scalar_prefetch=4,
        grid=(NI, NITEMS),
        in_specs=[
            pl.BlockSpec((TK, H), lambda i, j, rb, e, lo, hi: (0, 0)),
            pl.BlockSpec((1, BI, H), lambda i, j, rb, e, lo, hi: (e[j], i, 0)),
            pl.BlockSpec((1, BI, H), lambda i, j, rb, e, lo, hi: (e[j], i, 0)),
            pl.BlockSpec((1, H, BI), lambda i, j, rb, e, lo, hi: (e[j], 0, i)),
        ],
        out_specs=pl.BlockSpec((TK, H), lambda i, j, rb, e, lo, hi: (0, 0)),
    )
    return pl.pallas_call(
        _ffn_body,
        grid_spec=grid_spec,
        out_shape=jax.ShapeDtypeStruct((TK, H), jnp.float32),
        compiler_params=pltpu.CompilerParams(
            vmem_limit_bytes=60 * 1024 * 1024),
    )(item_rb, item_e, item_lo, item_hi, xs, w1, w3, w2)


def _combine_body(z0_ref, z1_ref, w_ref, out_ref):
    w0 = w_ref[:, 0:1]
    w1 = w_ref[:, 1:2]
    out_ref[...] = w0 * z0_ref[...] + w1 * z1_ref[...]


def _combine(z, w_pad):
    nblk = T // BM
    return pl.pallas_call(
        _combine_body,
        grid=(nblk,),
        in_specs=[
            pl.BlockSpec((BM, H), lambda i: (i, 0)),
            pl.BlockSpec((BM, H), lambda i, _n=nblk: (i + _n, 0)),
            pl.BlockSpec((BM, 128), lambda i: (i, 0)),
        ],
        out_specs=pl.BlockSpec((BM, H), lambda i: (i, 0)),
        out_shape=jax.ShapeDtypeStruct((T, H), jnp.float32),
    )(z, z, w_pad)


def kernel(hidden_states, Wg, W1, W3, W2):
    x = hidden_states.reshape(T, H)
    idx_pad, w_pad = _router(x, Wg)
    i1, i2 = idx_pad[:, 0], idx_pad[:, 1]
    sorted_t, pos0, pos1, item_rb, item_e, item_lo, item_hi = \
        _plan_dispatch(i1, i2)
    xs = _sc_gather_rows(x, sorted_t)
    y_sorted = _ffn(item_rb, item_e, item_lo, item_hi, xs, W1, W3, W2)
    z = _sc_gather_rows(y_sorted, jnp.concatenate([pos0, pos1]))
    out = _combine(z, w_pad)
    return out.reshape(hidden_states.shape)
